# baseline (speedup 1.0000x reference)
import functools

import jax
import jax.numpy as jnp
from jax import lax
from jax.experimental import pallas as pl
from jax.experimental.pallas import tpu as pltpu
from jax.experimental.pallas import tpu_sc as plsc

_DIM = 16
_BATCH = 16384
_NP = 2 * _BATCH
_NC = 2
_NS = 16
_NW = _NC * _NS
_WENT = 1024
_NWIN = 976
_SCAN_END = _NWIN * _WENT
_REMN = 1000000 - _SCAN_END
_WL = 2048
_SENT = 2147483647
_DUMP = _NP
_STAGE = _NP + 8
_EDGES_PER_W = _BATCH // _NW
_RING = 256


def _full(v):
    return jnp.full((_DIM,), v, jnp.int32)


def _pc(m):
    c = plsc.all_reduce_population_count(m)
    return lax.squeeze(lax.slice(c, (0,), (1,)), (0,))


def _sc(v, e):
    return lax.squeeze(lax.slice(v, (e,), (e + 1,)), (0,))


def _scatter_body(idx_hbm, tab_hbm, rem_hbm, stage_hbm,
                  idx_v, wl_r, wl_p, dn_r, dn_p, stacks, rem_v,
                  rowbuf, lineidx, sems, semf):
    wid = lax.axis_index("s") * _NC + lax.axis_index("c")
    iota = lax.iota(jnp.int32, _DIM)

    pltpu.sync_copy(idx_hbm, idx_v)
    pltpu.sync_copy(rem_hbm, rem_v)

    def filt(i, cnt):
        for t in range(8):
            rv = idx_v[i, pl.ds(t * _DIM, _DIM)]
            pv = i * 128 + t * _DIM + iota
            win = lax.shift_right_logical(rv, _full(10))
            own = (win & _full(31)) == wid
            remm = rv >= _SCAN_END
            m = jnp.where(remm, wid == 0, own)
            plsc.store_compressed(wl_r.at[pl.ds(cnt, _DIM)], rv, mask=m)
            plsc.store_compressed(wl_p.at[pl.ds(cnt, _DIM)], pv, mask=m)
            cnt = cnt + _pc(m)
        return cnt

    cnt = lax.fori_loop(0, _NP // 128, filt, jnp.int32(0))
    nwl = lax.div(cnt + (_DIM - 1), jnp.int32(_DIM))

    for h in range(2):
        for t in range(8):
            lineidx[h, pl.ds(t * _DIM, _DIM)] = _full(_DUMP)

    def fire(w, b):
        for blk in range(8):
            for g in range(2):
                pltpu.async_copy(
                    tab_hbm.at[pl.ds(8 * g, 8),
                               pl.ds(pl.multiple_of((w * 8 + blk) * 128, 128),
                                     128)],
                    stacks[b].at[pl.ds(blk * 16 + g * 8, 8), :],
                    sems[b])

    def drain(b):
        for _ in range(16):
            pltpu.make_async_copy(
                tab_hbm.at[pl.ds(0, 8), pl.ds(0, 128)],
                stacks[b].at[pl.ds(0, 8), :], sems[b]).wait()

    def flush(fcnt_old):
        half = lax.rem(lax.div(fcnt_old, jnp.int32(128)), jnp.int32(2))
        pltpu.async_copy(
            rowbuf.at[pl.ds(half * 128, 128), :],
            stage_hbm.at[lineidx.at[half]],
            semf).wait()
        for t in range(8):
            lineidx[half, pl.ds(t * _DIM, _DIM)] = _full(_DUMP)

    def process(wsel, buf_is_rem, buf, fcnt):
        def dense(vd, wcnt):
            rv = wl_r[pl.ds(vd * _DIM, _DIM)]
            pv = wl_p[pl.ds(vd * _DIM, _DIM)]
            valid = (vd * _DIM + iota) < cnt
            if buf_is_rem:
                m = (rv >= _SCAN_END) & valid
            else:
                m = (lax.shift_right_logical(rv, _full(10)) == wsel) & valid
            plsc.store_compressed(dn_r.at[pl.ds(wcnt, _DIM)], rv, mask=m)
            plsc.store_compressed(dn_p.at[pl.ds(wcnt, _DIM)], pv, mask=m)
            return wcnt + _pc(m)

        wcnt = lax.fori_loop(0, nwl, dense, jnp.int32(0))

        def emit(vd, fcnt):
            rv = dn_r[pl.ds(vd * _DIM, _DIM)]
            pv = dn_p[pl.ds(vd * _DIM, _DIM)]
            valid = (vd * _DIM + iota) < wcnt
            if buf_is_rem:
                rloc = rv - _SCAN_END
            else:
                rloc = rv - wsel * _WENT
            line = jnp.where(valid, pv, _full(_DUMP))
            halfv = lax.rem(
                lax.div(fcnt + iota, _full(128)), _full(2))
            slots7 = (fcnt + iota) & _full(127)
            plsc.store_scatter(lineidx, [halfv, slots7], line)
            if buf_is_rem:
                brow = jnp.minimum(
                    lax.shift_right_logical(rloc, _full(3)), _full(71))
                blane = lax.shift_left(rloc & _full(7), _full(4))
            else:
                blk = lax.shift_right_logical(rloc, _full(7)) & _full(7)
                lane = rloc & _full(127)
            for e in range(_DIM):
                if buf_is_rem:
                    br = _sc(brow, e)
                    bl = _sc(blane, e)
                    col = plsc.load_gather(
                        buf, [_full(0) + br, bl + iota])
                else:
                    be = _sc(blk, e)
                    le = _sc(lane, e)
                    col = plsc.load_gather(
                        buf, [be * 16 + iota, _full(0) + le])
                slot = lax.rem(fcnt + e, jnp.int32(_RING))
                rowbuf[slot, pl.ds(0, _DIM)] = col
            fcnt_new = fcnt + _DIM
            crossed = lax.div(fcnt_new, jnp.int32(128)) > lax.div(
                fcnt, jnp.int32(128))

            @pl.when(crossed)
            def _():
                flush(fcnt)

            return fcnt_new

        ndn = lax.div(wcnt + (_DIM - 1), jnp.int32(_DIM))
        return lax.fori_loop(0, ndn, emit, fcnt)

    njw = lax.div(jnp.int32(_NWIN - 1) - wid, jnp.int32(32)) + 1
    fire(wid, 0)

    def wloop(j, fcnt):
        w = wid + 32 * j

        def go(bi):
            def br(fc):
                drain(bi)

                @pl.when(j + 1 < njw)
                def _():
                    fire(w + 32, 1 - bi)

                return process(w, False, stacks[bi], fc)

            return br

        return lax.cond(lax.rem(j, jnp.int32(2)) == 0, go(0), go(1), fcnt)

    fcnt = lax.fori_loop(0, njw, wloop, jnp.int32(0))

    fcnt = lax.cond(wid == 0,
                    lambda f: process(0, True, rem_v, f),
                    lambda f: f, fcnt)

    flush(fcnt)
    flush(fcnt + 128)


def _rsqrt_scale(x):
    i = lax.bitcast_convert_type(x, jnp.int32)
    i = jnp.int32(0x5F3759DF) - lax.shift_right_arithmetic(i, _full(1))
    y = lax.bitcast_convert_type(i, jnp.float32)
    for _ in range(3):
        y = y * (1.5 - 0.5 * x * y * y)
    return jnp.where(x > 1.0, y, jnp.full((_DIM,), 1.0, jnp.float32))


def _compute_body(stage_hbm, out_hbm, bufs, probs_v, sems):
    wid = lax.axis_index("s") * _NC + lax.axis_index("c")
    lane = lax.iota(jnp.int32, _DIM)
    pbase = wid * 2 * _EDGES_PER_W

    def fire(k, b):
        pltpu.async_copy(stage_hbm.at[pl.ds(pbase + k * 128, 128), :],
                         bufs[b], sems[b])

    def drain(b):
        pltpu.make_async_copy(stage_hbm.at[pl.ds(0, 128), :], bufs[b],
                              sems[b]).wait()

    fire(0, 0)
    fire(1, 1)

    for k in range(8):
        b = k % 2
        buf = bufs[b]
        drain(b)

        def group(g, carry):
            base = g * (2 * _DIM)
            ss = jnp.zeros((_DIM,), jnp.float32)
            dd = jnp.zeros((_DIM,), jnp.float32)
            sd = jnp.zeros((_DIM,), jnp.float32)
            for e in range(_DIM):
                s = buf[base + 2 * e, pl.ds(0, _DIM)]
                d = buf[base + 2 * e + 1, pl.ds(0, _DIM)]
                m = lane == e
                ss = jnp.where(m, jnp.sum(s * s), ss)
                dd = jnp.where(m, jnp.sum(d * d), dd)
                sd = jnp.where(m, jnp.sum(s * d), sd)
            prod = sd * _rsqrt_scale(ss) * _rsqrt_scale(dd)
            probs_v[pl.ds(k * 64 + g * _DIM, _DIM)] = (
                1.0 / (1.0 + jnp.exp(-prod)))
            return carry

        lax.fori_loop(0, 4, group, 0)
        if k + 2 < 8:
            fire(k + 2, b)

    pltpu.sync_copy(probs_v,
                    out_hbm.at[pl.ds(wid * _EDGES_PER_W, _EDGES_PER_W)])


@jax.jit
def _scan_decoder(idx2, tabt, rem2):
    mesh = plsc.VectorSubcoreMesh(core_axis_name="c", subcore_axis_name="s")
    stage = pl.kernel(
        _scatter_body,
        mesh=mesh,
        compiler_params=pltpu.CompilerParams(needs_layout_passes=False),
        out_type=jax.ShapeDtypeStruct((_STAGE, 128), jnp.float32),
        scratch_types=[
            pltpu.VMEM((_NP // 128, 128), jnp.int32),
            pltpu.VMEM((_WL,), jnp.int32),
            pltpu.VMEM((_WL,), jnp.int32),
            pltpu.VMEM((128,), jnp.int32),
            pltpu.VMEM((128,), jnp.int32),
            [pltpu.VMEM((128, 128), jnp.float32) for _ in range(2)],
            pltpu.VMEM((_REMN // 8, 128), jnp.float32),
            pltpu.VMEM((_RING, 128), jnp.float32),
            pltpu.VMEM((2, 128), jnp.int32),
            [pltpu.SemaphoreType.DMA for _ in range(2)],
            pltpu.SemaphoreType.DMA,
        ],
    )(idx2, tabt, rem2)
    return pl.kernel(
        _compute_body,
        mesh=mesh,
        compiler_params=pltpu.CompilerParams(needs_layout_passes=False),
        out_type=jax.ShapeDtypeStruct((_BATCH,), jnp.float32),
        scratch_types=[
            [pltpu.VMEM((128, 128), jnp.float32) for _ in range(2)],
            pltpu.VMEM((_EDGES_PER_W,), jnp.float32),
            [pltpu.SemaphoreType.DMA for _ in range(2)],
        ],
    )(stage)


def kernel(edges, table):
    idx2 = edges.astype(jnp.int32).reshape(_NP // 128, 128)
    tabt = table.T
    rem2 = table[_SCAN_END:, :].reshape(_REMN // 8, 128)
    return _scan_decoder(idx2, tabt, rem2)


# speedup vs baseline: 1.5069x; 1.5069x over previous
import functools

import jax
import jax.numpy as jnp
from jax import lax
from jax.experimental import pallas as pl
from jax.experimental.pallas import tpu as pltpu
from jax.experimental.pallas import tpu_sc as plsc

_DIM = 16
_BATCH = 16384
_NC = 2
_NS = 16
_NW = _NC * _NS
_EDGES_PER_W = _BATCH // _NW
_ROWS_PER_W = 2 * _EDGES_PER_W
_CHUNK = 128
_NCHUNK = _ROWS_PER_W // _CHUNK
_GROUPS = _EDGES_PER_W // _DIM


def _rsqrt_scale(x):
    i = lax.bitcast_convert_type(x, jnp.int32)
    i = jnp.int32(0x5F3759DF) - lax.shift_right_arithmetic(
        i, jnp.full((_DIM,), 1, jnp.int32))
    y = lax.bitcast_convert_type(i, jnp.float32)
    for _ in range(3):
        y = y * (1.5 - 0.5 * x * y * y)
    return jnp.where(x > 1.0, y, jnp.full((_DIM,), 1.0, jnp.float32))


def _decoder_body(idx_hbm, table_hbm, out_hbm, idx_v, rows_v, probs_v, sem):
    wid = lax.axis_index("s") * _NC + lax.axis_index("c")

    pltpu.sync_copy(idx_hbm.at[pl.ds(wid * _NCHUNK, _NCHUNK), :], idx_v)

    copies = []
    for k in range(_NCHUNK):
        copies.append(
            pltpu.async_copy(
                table_hbm.at[idx_v.at[k]],
                rows_v.at[pl.ds(k * _CHUNK, _CHUNK), :],
                sem,
            ))
    for cp in copies:
        cp.wait()

    lane = lax.iota(jnp.int32, _DIM)

    def group(g, carry):
        base = g * (2 * _DIM)
        ss = jnp.zeros((_DIM,), jnp.float32)
        dd = jnp.zeros((_DIM,), jnp.float32)
        sd = jnp.zeros((_DIM,), jnp.float32)
        for e in range(_DIM):
            s = rows_v[base + 2 * e, :]
            d = rows_v[base + 2 * e + 1, :]
            m = lane == e
            ss = jnp.where(m, jnp.sum(s * s), ss)
            dd = jnp.where(m, jnp.sum(d * d), dd)
            sd = jnp.where(m, jnp.sum(s * d), sd)
        prod = sd * _rsqrt_scale(ss) * _rsqrt_scale(dd)
        probs_v[pl.ds(g * _DIM, _DIM)] = 1.0 / (1.0 + jnp.exp(-prod))
        return carry

    lax.fori_loop(0, _GROUPS, group, 0)

    pltpu.sync_copy(probs_v, out_hbm.at[pl.ds(wid * _EDGES_PER_W, _EDGES_PER_W)])


@jax.jit
def _decoder(idx2, table):
    mesh = plsc.VectorSubcoreMesh(core_axis_name="c", subcore_axis_name="s")
    return pl.kernel(
        _decoder_body,
        mesh=mesh,
        compiler_params=pltpu.CompilerParams(
            needs_layout_passes=False, use_tc_tiling_on_sc=False),
        out_type=jax.ShapeDtypeStruct((_BATCH,), jnp.float32),
        scratch_types=[
            pltpu.VMEM((_NCHUNK, _CHUNK), jnp.int32),
            pltpu.VMEM((_ROWS_PER_W, _DIM), jnp.float32),
            pltpu.VMEM((_EDGES_PER_W,), jnp.float32),
            pltpu.SemaphoreType.DMA,
        ],
    )(idx2, table)


def kernel(edges, table):
    idx2 = edges.astype(jnp.int32).reshape(_BATCH * 2 // _CHUNK, _CHUNK)
    return _decoder(idx2, table)
